# Initial kernel scaffold; baseline (speedup 1.0000x reference)
import jax
import jax.numpy as jnp
from jax.experimental import pallas as pl


def kernel(x, edge_index, batch_index, descriptors, gin_w, gin_b, gat_w, att_src, att_dst, gat_b, sage_wl, sage_bl, sage_wr, fc1_w, fc1_b, aff1_w, aff1_b, aff2_w, aff2_b):
    raise NotImplementedError("write your pallas kernel here")


# speedup vs baseline: 1.0360x; 1.0360x over previous
import functools
import jax
import jax.numpy as jnp
from jax.experimental import pallas as pl
from jax.experimental.pallas import tpu as pltpu

HIDDEN = 128
HEADS = 2
DESC = 256


def _gelu(v):
    return jax.nn.gelu(v, approximate=False)


def _erf(z):
    s = jnp.sign(z)
    z = jnp.abs(z)
    t = 1.0 / (1.0 + 0.3275911 * z)
    poly = t * (0.254829592 + t * (-0.284496736 + t * (1.421413741
           + t * (-1.453152027 + t * 1.061405429))))
    return s * (1.0 - poly * jnp.exp(-z * z))


def _gelu_p(v):
    return 0.5 * v * (1.0 + _erf(v * 0.7071067811865476))


def _gin_body(xa_ref, w_ref, b_ref, o_ref):
    o_ref[...] = _gelu_p(
        jnp.dot(xa_ref[...], w_ref[...], preferred_element_type=jnp.float32)
        + b_ref[...]
    )


def _gin_dense(xa16, w16, b):
    N = xa16.shape[0]
    BN = 2000
    return pl.pallas_call(
        _gin_body,
        grid=(N // BN,),
        in_specs=[
            pl.BlockSpec((BN, 16), lambda i: (i, 0)),
            pl.BlockSpec((16, HIDDEN), lambda i: (0, 0)),
            pl.BlockSpec((HIDDEN,), lambda i: (0,)),
        ],
        out_specs=pl.BlockSpec((BN, HIDDEN), lambda i: (i, 0)),
        out_shape=jax.ShapeDtypeStruct((N, HIDDEN), jnp.float32),
    )(xa16, w16, b)


def kernel(x, edge_index, batch_index, descriptors, gin_w, gin_b, gat_w,
           att_src, att_dst, gat_b, sage_wl, sage_bl, sage_wr, fc1_w, fc1_b,
           aff1_w, aff1_b, aff2_w, aff2_b):
    N = x.shape[0]
    src = edge_index[0].astype(jnp.int32)
    dst = edge_index[1].astype(jnp.int32)
    batch_index = batch_index.astype(jnp.int32)

    agg = jnp.zeros((N, x.shape[1]), dtype=x.dtype).at[dst].add(x[src])
    xa = x + agg
    xa16 = jnp.pad(xa, ((0, 0), (0, 16 - xa.shape[1])))
    w16 = jnp.pad(gin_w, ((0, 16 - gin_w.shape[0]), (0, 0)))
    h = _gin_dense(xa16, w16, gin_b)

    hp = (h @ gat_w).reshape(N, HEADS, HIDDEN)
    a_src = (hp * att_src[None, :, :]).sum(-1)
    a_dst = (hp * att_dst[None, :, :]).sum(-1)
    e = a_src[src] + a_dst[dst]
    e = jax.nn.leaky_relu(e, negative_slope=0.2)
    ee = jnp.exp(e)
    esum = jax.ops.segment_sum(ee, dst, num_segments=N)
    alpha = ee / (esum[dst] + 1e-16)
    msg = hp[src] * alpha[:, :, None]
    out = jax.ops.segment_sum(msg, dst, num_segments=N)
    h2 = _gelu(out.mean(axis=1) + gat_b)

    nsum = jax.ops.segment_sum(h2[src], dst, num_segments=N)
    ncnt = jax.ops.segment_sum(jnp.ones((src.shape[0],), jnp.float32), dst,
                               num_segments=N)
    nmean = nsum / jnp.maximum(ncnt, 1.0)[:, None]
    h3 = _gelu(nmean @ sage_wl + sage_bl + h2 @ sage_wr)

    G = descriptors.shape[0]
    gmax = jax.ops.segment_max(h3, batch_index, num_segments=G)
    gmax = jnp.where(jnp.isfinite(gmax), gmax, 0.0)
    gsum = jax.ops.segment_sum(h3, batch_index, num_segments=G)
    gcnt = jax.ops.segment_sum(jnp.ones((N,), jnp.float32), batch_index,
                               num_segments=G)
    gmean = gsum / jnp.maximum(gcnt, 1.0)[:, None]
    combined = jnp.concatenate([gmax, gmean, descriptors], axis=1)
    s = _gelu(combined @ fc1_w + fc1_b)
    a = _gelu(s @ aff1_w + aff1_b)
    return a @ aff2_w + aff2_b


# speedup vs baseline: 5.4844x; 5.2939x over previous
import functools
import jax
import jax.numpy as jnp
from jax import lax
from jax.experimental import pallas as pl
from jax.experimental.pallas import tpu as pltpu
from jax.experimental.pallas import tpu_sc as plsc

HIDDEN = 128
HEADS = 2
DESC = 256

NC = 2
NS = 16
NW = NC * NS
L = 16

N_PAD = 100032
ROWS_PER_SUB = N_PAD // NS
ZROWS = 521
CH = 1024
SUB = 128
NSUB = CH // SUB
E_PAD = NW * 50 * CH
NCHUNK = 50

_mesh = plsc.VectorSubcoreMesh(core_axis_name="c", subcore_axis_name="s")


def _zero_acc(zbuf_v, acc_sh, sid):
    def zb(i, c):
        zbuf_v[i] = jnp.zeros((L,), jnp.float32)
        return c
    lax.fori_loop(0, ZROWS, zb, 0)
    for k in range(ROWS_PER_SUB // ZROWS):
        pltpu.sync_copy(zbuf_v,
                        acc_sh.at[pl.ds(sid * ROWS_PER_SUB + k * ZROWS, ZROWS)])


def _gss_body(weighted, table_h, src_h, dst_h, wb_h, out_h, *scr):
    sbufs = scr[0:NSUB]
    dbufs = scr[NSUB:2 * NSUB]
    rows_v, w_v, zbuf_v, acc_sh, sem = scr[2 * NSUB:]
    cid = lax.axis_index("c")
    sid = lax.axis_index("s")
    wid = sid * NC + cid
    _zero_acc(zbuf_v, acc_sh, sid)
    plsc.subcore_barrier()

    def chunk(t, c):
        j = wid * NCHUNK + t
        base = j * CH
        for k in range(NSUB):
            pltpu.sync_copy(src_h.at[pl.ds(base + k * SUB, SUB)], sbufs[k])
            pltpu.sync_copy(dst_h.at[pl.ds(base + k * SUB, SUB)], dbufs[k])
        for k in range(NSUB):
            pltpu.async_copy(table_h.at[sbufs[k]],
                             rows_v.at[pl.ds(k * SUB, SUB)], sem).wait()
        if weighted:
            pltpu.sync_copy(wb_h.at[pl.ds(base, CH)], w_v)

            def mul(i, c2):
                rows_v[i] = rows_v[i] * w_v[i]
                return c2
            lax.fori_loop(0, CH, mul, 0)
        for k in range(NSUB):
            pltpu.sync_copy(rows_v.at[pl.ds(k * SUB, SUB)],
                            acc_sh.at[dbufs[k]], add=True)
        return c
    lax.fori_loop(0, NCHUNK, chunk, 0)
    plsc.subcore_barrier()
    pltpu.sync_copy(acc_sh.at[pl.ds(sid * ROWS_PER_SUB, ROWS_PER_SUB)],
                    out_h.at[pl.ds(cid * N_PAD + sid * ROWS_PER_SUB,
                                   ROWS_PER_SUB)])


def _make_gss(weighted):
    scratch = (
        [pltpu.VMEM((SUB,), jnp.int32) for _ in range(2 * NSUB)]
        + [
            pltpu.VMEM((CH, L), jnp.float32),
            pltpu.VMEM((CH, L), jnp.float32),
            pltpu.VMEM((ZROWS, L), jnp.float32),
            pltpu.VMEM_SHARED((N_PAD, L), jnp.float32),
            pltpu.SemaphoreType.DMA,
        ]
    )
    params = pltpu.CompilerParams(use_tc_tiling_on_sc=False)
    if weighted:
        def f(table, src3, dst3, wb):
            return pl.kernel(
                functools.partial(_gss_body, True),
                mesh=_mesh,
                out_type=jax.ShapeDtypeStruct((NC * N_PAD, L), jnp.float32),
                scratch_types=scratch,
                compiler_params=params,
            )(table, src3, dst3, wb)
    else:
        def f(table, src3, dst3):
            dummy = jnp.zeros((8, L), jnp.float32)
            return pl.kernel(
                functools.partial(_gss_body, False),
                mesh=_mesh,
                out_type=jax.ShapeDtypeStruct((NC * N_PAD, L), jnp.float32),
                scratch_types=scratch,
                compiler_params=params,
            )(table, src3, dst3, dummy)
    return f


_gss_w = _make_gss(True)
_gss_u = _make_gss(False)


def _att_body(asrc_h, adst_h, src_h, dst_h, eeraw_h, *scr):
    sbufs = scr[0:NSUB]
    dbufs = scr[NSUB:2 * NSUB]
    ra_v, rb_v, ee_v, sem = scr[2 * NSUB:]
    cid = lax.axis_index("c")
    sid = lax.axis_index("s")
    wid = sid * NC + cid

    def chunk(t, c):
        j = wid * NCHUNK + t
        base = j * CH
        for k in range(NSUB):
            pltpu.sync_copy(src_h.at[pl.ds(base + k * SUB, SUB)], sbufs[k])
            pltpu.sync_copy(dst_h.at[pl.ds(base + k * SUB, SUB)], dbufs[k])
        for k in range(NSUB):
            pltpu.async_copy(asrc_h.at[sbufs[k]],
                             ra_v.at[pl.ds(k * SUB, SUB)], sem).wait()
            pltpu.async_copy(adst_h.at[dbufs[k]],
                             rb_v.at[pl.ds(k * SUB, SUB)], sem).wait()

        def edge(i, c2):
            tt = ra_v[i] + rb_v[i]
            tt = jnp.maximum(tt, 0.0) + 0.2 * jnp.minimum(tt, 0.0)
            ee_v[i] = jnp.exp(tt)
            return c2
        lax.fori_loop(0, CH, edge, 0)
        pltpu.sync_copy(ee_v, eeraw_h.at[pl.ds(base, CH)])
        return c
    lax.fori_loop(0, NCHUNK, chunk, 0)


_att_call = pl.kernel(
    _att_body,
    mesh=_mesh,
    out_type=jax.ShapeDtypeStruct((E_PAD, L), jnp.float32),
    scratch_types=(
        [pltpu.VMEM((SUB,), jnp.int32) for _ in range(2 * NSUB)]
        + [
            pltpu.VMEM((CH, L), jnp.float32),
            pltpu.VMEM((CH, L), jnp.float32),
            pltpu.VMEM((CH, L), jnp.float32),
            pltpu.SemaphoreType.DMA,
        ]
    ),
    compiler_params=pltpu.CompilerParams(use_tc_tiling_on_sc=False),
)


def _gmul_body(table_h, src_h, wb_h, rows_out_h, *scr):
    sbufs = scr[0:NSUB]
    rows_v, w_v, sem = scr[NSUB:]
    cid = lax.axis_index("c")
    sid = lax.axis_index("s")
    wid = sid * NC + cid

    def chunk(t, c):
        j = wid * NCHUNK + t
        base = j * CH
        for k in range(NSUB):
            pltpu.sync_copy(src_h.at[pl.ds(base + k * SUB, SUB)], sbufs[k])
        for k in range(NSUB):
            pltpu.async_copy(table_h.at[sbufs[k]],
                             rows_v.at[pl.ds(k * SUB, SUB)], sem).wait()
        pltpu.sync_copy(wb_h.at[pl.ds(base, CH)], w_v)

        def mul(i, c2):
            rows_v[i] = rows_v[i] * w_v[i]
            return c2
        lax.fori_loop(0, CH, mul, 0)
        pltpu.sync_copy(rows_v, rows_out_h.at[pl.ds(base, CH)])
        return c
    lax.fori_loop(0, NCHUNK, chunk, 0)


_gmul_call = pl.kernel(
    _gmul_body,
    mesh=_mesh,
    out_type=jax.ShapeDtypeStruct((E_PAD, L), jnp.float32),
    scratch_types=(
        [pltpu.VMEM((SUB,), jnp.int32) for _ in range(NSUB)]
        + [
            pltpu.VMEM((CH, L), jnp.float32),
            pltpu.VMEM((CH, L), jnp.float32),
            pltpu.SemaphoreType.DMA,
        ]
    ),
    compiler_params=pltpu.CompilerParams(use_tc_tiling_on_sc=False),
)


def _scat_body(vals_h, dst_h, out_h, *scr):
    dbufs = scr[0:NSUB]
    rows_v, zbuf_v, acc_sh, sem = scr[NSUB:]
    cid = lax.axis_index("c")
    sid = lax.axis_index("s")
    wid = sid * NC + cid
    _zero_acc(zbuf_v, acc_sh, sid)
    plsc.subcore_barrier()

    def chunk(t, c):
        j = wid * NCHUNK + t
        base = j * CH
        for k in range(NSUB):
            pltpu.sync_copy(dst_h.at[pl.ds(base + k * SUB, SUB)], dbufs[k])
        pltpu.sync_copy(vals_h.at[pl.ds(base, CH)], rows_v)
        for k in range(NSUB):
            pltpu.sync_copy(rows_v.at[pl.ds(k * SUB, SUB)],
                            acc_sh.at[dbufs[k]], add=True)
        return c
    lax.fori_loop(0, NCHUNK, chunk, 0)
    plsc.subcore_barrier()
    pltpu.sync_copy(acc_sh.at[pl.ds(sid * ROWS_PER_SUB, ROWS_PER_SUB)],
                    out_h.at[pl.ds(cid * N_PAD + sid * ROWS_PER_SUB,
                                   ROWS_PER_SUB)])


_scat_call = pl.kernel(
    _scat_body,
    mesh=_mesh,
    out_type=jax.ShapeDtypeStruct((NC * N_PAD, L), jnp.float32),
    scratch_types=(
        [pltpu.VMEM((SUB,), jnp.int32) for _ in range(NSUB)]
        + [
            pltpu.VMEM((CH, L), jnp.float32),
            pltpu.VMEM((ZROWS, L), jnp.float32),
            pltpu.VMEM_SHARED((N_PAD, L), jnp.float32),
            pltpu.SemaphoreType.DMA,
        ]
    ),
    compiler_params=pltpu.CompilerParams(use_tc_tiling_on_sc=False),
)


def _bc_body(x_ref, o0_ref, o1_ref):
    o0_ref[...] = jnp.broadcast_to(x_ref[:, 0:1], x_ref.shape)
    o1_ref[...] = jnp.broadcast_to(x_ref[:, 1:2], x_ref.shape)


def _bcast01(eeraw):
    BN = 4096
    return pl.pallas_call(
        _bc_body,
        grid=(E_PAD // BN,),
        in_specs=[pl.BlockSpec((BN, L), lambda i: (i, 0))],
        out_specs=[pl.BlockSpec((BN, L), lambda i: (i, 0)),
                   pl.BlockSpec((BN, L), lambda i: (i, 0))],
        out_shape=[jax.ShapeDtypeStruct((E_PAD, L), jnp.float32),
                   jax.ShapeDtypeStruct((E_PAD, L), jnp.float32)],
    )(eeraw)


def _pad_nodes(a):
    return jnp.pad(a, ((0, N_PAD - a.shape[0]), (0, 16 - a.shape[1])))


def _gelu(v):
    return jax.nn.gelu(v, approximate=False)


def _erf(z):
    s = jnp.sign(z)
    z = jnp.abs(z)
    t = 1.0 / (1.0 + 0.3275911 * z)
    poly = t * (0.254829592 + t * (-0.284496736 + t * (1.421413741
           + t * (-1.453152027 + t * 1.061405429))))
    return s * (1.0 - poly * jnp.exp(-z * z))


def _gelu_p(v):
    return 0.5 * v * (1.0 + _erf(v * 0.7071067811865476))


def _gin_body(xa_ref, w_ref, b_ref, o_ref):
    o_ref[...] = _gelu_p(
        jnp.dot(xa_ref[...], w_ref[...], preferred_element_type=jnp.float32)
        + b_ref[...]
    )


def _gin_dense(xa16, w16, b):
    N = xa16.shape[0]
    BN = 2000
    return pl.pallas_call(
        _gin_body,
        grid=(N // BN,),
        in_specs=[
            pl.BlockSpec((BN, 16), lambda i: (i, 0)),
            pl.BlockSpec((16, HIDDEN), lambda i: (0, 0)),
            pl.BlockSpec((HIDDEN,), lambda i: (0,)),
        ],
        out_specs=pl.BlockSpec((BN, HIDDEN), lambda i: (i, 0)),
        out_shape=jax.ShapeDtypeStruct((N, HIDDEN), jnp.float32),
    )(xa16, w16, b)


def kernel(x, edge_index, batch_index, descriptors, gin_w, gin_b, gat_w,
           att_src, att_dst, gat_b, sage_wl, sage_bl, sage_wr, fc1_w, fc1_b,
           aff1_w, aff1_b, aff2_w, aff2_b):
    N = x.shape[0]
    src = edge_index[0].astype(jnp.int32)
    dst = edge_index[1].astype(jnp.int32)
    batch_index = batch_index.astype(jnp.int32)

    E = edge_index.shape[1]
    pad_idx = jnp.full((E_PAD - E,), N, jnp.int32)
    src3 = jnp.concatenate([src, pad_idx])
    dst3 = jnp.concatenate([dst, pad_idx])
    x16 = _pad_nodes(x)
    aggp = _gss_u(x16, src3, dst3)
    xa16 = x16[:N] + aggp[:N] + aggp[N_PAD:N_PAD + N]
    w16 = jnp.pad(gin_w, ((0, 16 - gin_w.shape[0]), (0, 0)))
    h = _gin_dense(xa16, w16, gin_b)

    hp = (h @ gat_w).reshape(N, HEADS, HIDDEN)
    a_src = (hp * att_src[None, :, :]).sum(-1)
    a_dst = (hp * att_dst[None, :, :]).sum(-1)
    eeraw = _att_call(_pad_nodes(a_src), _pad_nodes(a_dst), src3, dst3)
    esump = _scat_call(eeraw, dst3)
    eeb0, eeb1 = _bcast01(eeraw)
    esum = esump[:N] + esump[N_PAD:N_PAD + N]
    ncnt = esum[:, 2]
    hpf = hp.reshape(N, HEADS * HIDDEN)
    chunks = []
    for c in range(16):
        tab = _pad_nodes(hpf[:, 16 * c:16 * (c + 1)])
        scaled = _gmul_call(tab, src3, eeb0 if c < 8 else eeb1)
        p = _scat_call(scaled, dst3)
        chunks.append(p[:N] + p[N_PAD:N_PAD + N])
    msum = jnp.concatenate(chunks, axis=1).reshape(N, HEADS, HIDDEN)
    out = msum / (esum[:, :HEADS, None] + 1e-16)
    h2 = _gelu(out.mean(axis=1) + gat_b)

    schunks = []
    for c in range(8):
        tab = _pad_nodes(h2[:, 16 * c:16 * (c + 1)])
        p = _gss_u(tab, src3, dst3)
        schunks.append(p[:N] + p[N_PAD:N_PAD + N])
    nsum = jnp.concatenate(schunks, axis=1)
    nmean = nsum / jnp.maximum(ncnt, 1.0)[:, None]
    h3 = _gelu(nmean @ sage_wl + sage_bl + h2 @ sage_wr)

    G = descriptors.shape[0]
    gmax = jax.ops.segment_max(h3, batch_index, num_segments=G)
    gmax = jnp.where(jnp.isfinite(gmax), gmax, 0.0)
    gsum = jax.ops.segment_sum(h3, batch_index, num_segments=G)
    gcnt = jax.ops.segment_sum(jnp.ones((N,), jnp.float32), batch_index,
                               num_segments=G)
    gmean = gsum / jnp.maximum(gcnt, 1.0)[:, None]
    combined = jnp.concatenate([gmax, gmean, descriptors], axis=1)
    s = _gelu(combined @ fc1_w + fc1_b)
    a = _gelu(s @ aff1_w + aff1_b)
    return a @ aff2_w + aff2_b
